# baseline (speedup 1.0000x reference)
import functools

import jax
import jax.numpy as jnp
from jax import lax
from jax.experimental import pallas as pl
from jax.experimental.pallas import tpu as pltpu
from jax.experimental.pallas import tpu_sc as plsc

N = 10000
D = 128
E = 320000

NC = 2
NS = 16
NW = NC * NS

K = 64
EPT = 10240
E_PAD = NW * EPT
CHUNKS = EPT // K
N_PAD = 10112
ROWS_PT = N_PAD // NS

_MM_BLOCK = 1000


def _fused_mm_body(p_ref0, p_ref1, b_ref, w1_ref, w2_ref, o_ref):
    h = jnp.maximum(
        jnp.dot(p_ref0[0] + p_ref1[0], w1_ref[...],
                preferred_element_type=jnp.float32) + b_ref[...], 0.0)
    o_ref[...] = jnp.dot(h, w2_ref[...], preferred_element_type=jnp.float32)


def _tc_fused_mms(p, b, w1, w2):
    return pl.pallas_call(
        _fused_mm_body,
        grid=(N // _MM_BLOCK,),
        in_specs=[
            pl.BlockSpec((1, _MM_BLOCK, D), lambda i: (0, i, 0)),
            pl.BlockSpec((1, _MM_BLOCK, D), lambda i: (1, i, 0)),
            pl.BlockSpec((1, D), lambda i: (0, 0)),
            pl.BlockSpec((D, D), lambda i: (0, 0)),
            pl.BlockSpec((D, D), lambda i: (0, 0)),
        ],
        out_specs=pl.BlockSpec((_MM_BLOCK, D), lambda i: (i, 0)),
        out_shape=jax.ShapeDtypeStruct((N, D), jnp.float32),
    )(p, p, b, w1, w2)


def _comb_body(p_ref0, p_ref1, b_ref, o_ref):
    o_ref[...] = p_ref0[0] + p_ref1[0] + b_ref[...]


def _tc_combine(p, b):
    return pl.pallas_call(
        _comb_body,
        grid=(N // _MM_BLOCK,),
        in_specs=[
            pl.BlockSpec((1, _MM_BLOCK, D), lambda i: (0, i, 0)),
            pl.BlockSpec((1, _MM_BLOCK, D), lambda i: (1, i, 0)),
            pl.BlockSpec((1, D), lambda i: (0, 0)),
        ],
        out_specs=pl.BlockSpec((_MM_BLOCK, D), lambda i: (i, 0)),
        out_shape=jax.ShapeDtypeStruct((N, D), jnp.float32),
    )(p, p, b)


@functools.partial(
    pl.kernel,
    out_type=jax.ShapeDtypeStruct((NC, N_PAD, D), jnp.float32),
    mesh=plsc.VectorSubcoreMesh(core_axis_name="c", subcore_axis_name="s"),
    scratch_types=[
        pltpu.VMEM((4, 2, K), jnp.int32),
        pltpu.VMEM((4, K, D), jnp.float32),
        pltpu.VMEM_SHARED((N_PAD, D), jnp.float32),
        [pltpu.SemaphoreType.DMA] * 4,
        [pltpu.SemaphoreType.DMA] * 4,
        [pltpu.SemaphoreType.DMA] * 4,
        pltpu.SemaphoreType.DMA,
    ],
)
def _sc_segment_sum(h_hbm, eidx_hbm, zeros_hbm, out_hbm,
                    slab, rows_v, accum, gsems, ssems, slsems, zsem):
    c = lax.axis_index("c")
    s = lax.axis_index("s")
    wid = s * NC + c

    zdesc = pltpu.async_copy(zeros_hbm.at[pl.ds(s * ROWS_PT, ROWS_PT)],
                             accum.at[pl.ds(s * ROWS_PT, ROWS_PT)], zsem)

    def gather(sl, p):
        pltpu.async_copy(h_hbm.at[slab.at[sl, 0]], rows_v.at[p], gsems[p])

    def wait_gather(sl, p):
        pltpu.make_async_copy(h_hbm.at[slab.at[sl, 0]], rows_v.at[p],
                              gsems[p]).wait()

    def scatter(sl, p):
        pltpu.async_copy(rows_v.at[p], accum.at[slab.at[sl, 1]], ssems[p],
                         add=True)

    def wait_scatter(sl, p):
        pltpu.make_async_copy(rows_v.at[p], accum.at[slab.at[sl, 1]],
                              ssems[p]).wait()

    def load_slab(j, sl):
        pltpu.async_copy(eidx_hbm.at[wid, j], slab.at[sl], slsems[sl])

    def wait_slab(j, sl):
        pltpu.make_async_copy(eidx_hbm.at[wid, j], slab.at[sl],
                              slsems[sl]).wait()

    pltpu.sync_copy(eidx_hbm.at[wid, 0], slab.at[0])
    gather(0, 0)
    load_slab(1, 1)
    zdesc.wait()
    plsc.subcore_barrier()

    def quad_body(t, _):
        for q in range(4):
            j = 4 * t + q

            @pl.when(j + 1 < CHUNKS)
            def _():
                wait_slab(j + 1, (q + 1) % 4)
                gather((q + 1) % 4, (q + 1) % 4)

            wait_gather(q, q)
            scatter(q, q)

            if q == 0:
                @pl.when(t > 0)
                def _():
                    wait_scatter(3, 3)
            else:
                wait_scatter(q - 1, q - 1)

            @pl.when(j + 2 < CHUNKS)
            def _():
                load_slab(j + 2, (q + 2) % 4)
        return 0

    lax.fori_loop(0, CHUNKS // 4, quad_body, 0)

    wait_scatter((CHUNKS - 1) % 4, (CHUNKS - 1) % 4)

    plsc.subcore_barrier()
    pltpu.sync_copy(accum.at[pl.ds(s * ROWS_PT, ROWS_PT)],
                    out_hbm.at[c, pl.ds(s * ROWS_PT, ROWS_PT)])


def kernel(x, edge_index, W1, b1, W2, b2):
    pad = E_PAD - E
    src_p = jnp.concatenate([edge_index[1], jnp.zeros((pad,), jnp.int32)])
    dst_p = jnp.concatenate([edge_index[0], jnp.full((pad,), N, jnp.int32)])
    eidx = jnp.stack([src_p.reshape(NW, CHUNKS, K),
                      dst_p.reshape(NW, CHUNKS, K)], axis=2)
    zeros = jnp.zeros((N_PAD, D), jnp.float32)
    b1r = b1.reshape(1, D)
    b2r = b2.reshape(1, D)

    p = _sc_segment_sum(x, eidx, zeros)
    h = _tc_fused_mms(p, b1r, W1, W2)
    q = _sc_segment_sum(h, eidx, zeros)
    return _tc_combine(q, b2r)


# speedup vs baseline: 3.6860x; 3.6860x over previous
import functools

import jax
import jax.numpy as jnp
from jax import lax
from jax.experimental import pallas as pl
from jax.experimental.pallas import tpu as pltpu
from jax.experimental.pallas import tpu_sc as plsc

N = 10000
D = 128
E = 320000

NC = 2
NS = 16
NW = NC * NS

K = 100
EPT = E // NW
CHUNKS = EPT // K
N_PAD = 10112
ROWS_PT = N_PAD // NS

_MM_BLOCK = 1000


def _fused_mm_body(p_ref0, p_ref1, b_ref, w1_ref, w2_ref, o_ref):
    h = jnp.maximum(
        jnp.dot(p_ref0[0] + p_ref1[0], w1_ref[...],
                preferred_element_type=jnp.float32) + b_ref[...], 0.0)
    o_ref[...] = jnp.dot(h, w2_ref[...], preferred_element_type=jnp.float32)


def _tc_fused_mms(p, b, w1, w2):
    return pl.pallas_call(
        _fused_mm_body,
        grid=(N // _MM_BLOCK,),
        in_specs=[
            pl.BlockSpec((1, _MM_BLOCK, D), lambda i: (0, i, 0)),
            pl.BlockSpec((1, _MM_BLOCK, D), lambda i: (1, i, 0)),
            pl.BlockSpec((1, D), lambda i: (0, 0)),
            pl.BlockSpec((D, D), lambda i: (0, 0)),
            pl.BlockSpec((D, D), lambda i: (0, 0)),
        ],
        out_specs=pl.BlockSpec((_MM_BLOCK, D), lambda i: (i, 0)),
        out_shape=jax.ShapeDtypeStruct((N, D), jnp.float32),
    )(p, p, b, w1, w2)


def _comb_body(p_ref0, p_ref1, b_ref, o_ref):
    o_ref[...] = p_ref0[0] + p_ref1[0] + b_ref[...]


def _tc_combine(p, b):
    return pl.pallas_call(
        _comb_body,
        grid=(N // _MM_BLOCK,),
        in_specs=[
            pl.BlockSpec((1, _MM_BLOCK, D), lambda i: (0, i, 0)),
            pl.BlockSpec((1, _MM_BLOCK, D), lambda i: (1, i, 0)),
            pl.BlockSpec((1, D), lambda i: (0, 0)),
        ],
        out_specs=pl.BlockSpec((_MM_BLOCK, D), lambda i: (i, 0)),
        out_shape=jax.ShapeDtypeStruct((N, D), jnp.float32),
    )(p, p, b)


@functools.partial(
    pl.kernel,
    out_type=jax.ShapeDtypeStruct((NC, N_PAD, D), jnp.float32),
    mesh=plsc.VectorSubcoreMesh(core_axis_name="c", subcore_axis_name="s"),
    scratch_types=[
        pltpu.VMEM((4, 2, K), jnp.int32),
        pltpu.VMEM((2, K, D), jnp.float32),
        pltpu.VMEM_SHARED((N_PAD, D), jnp.float32),
        [pltpu.SemaphoreType.DMA] * 2,
        [pltpu.SemaphoreType.DMA] * 2,
        [pltpu.SemaphoreType.DMA] * 4,
        pltpu.SemaphoreType.DMA,
    ],
)
def _sc_segment_sum(h_hbm, eidx_hbm, zeros_hbm, out_hbm,
                    slab, rows_v, accum, gsems, ssems, slsems, zsem):
    c = lax.axis_index("c")
    s = lax.axis_index("s")
    wid = s * NC + c

    zdesc = pltpu.async_copy(zeros_hbm.at[pl.ds(s * ROWS_PT, ROWS_PT)],
                             accum.at[pl.ds(s * ROWS_PT, ROWS_PT)], zsem)

    def gather(sl, p):
        pltpu.async_copy(h_hbm.at[slab.at[sl, 0]], rows_v.at[p], gsems[p])

    def wait_gather(sl, p):
        pltpu.make_async_copy(h_hbm.at[slab.at[sl, 0]], rows_v.at[p],
                              gsems[p]).wait()

    def scatter(sl, p):
        pltpu.async_copy(rows_v.at[p], accum.at[slab.at[sl, 1]], ssems[p],
                         add=True)

    def wait_scatter(sl, p):
        pltpu.make_async_copy(rows_v.at[p], accum.at[slab.at[sl, 1]],
                              ssems[p]).wait()

    def load_slab(j, sl):
        pltpu.async_copy(eidx_hbm.at[wid, j], slab.at[sl], slsems[sl])

    def wait_slab(j, sl):
        pltpu.make_async_copy(eidx_hbm.at[wid, j], slab.at[sl],
                              slsems[sl]).wait()

    pltpu.sync_copy(eidx_hbm.at[wid, 0], slab.at[0])
    gather(0, 0)
    load_slab(1, 1)
    zdesc.wait()
    plsc.subcore_barrier()

    def quad_body(t, _):
        for q in range(4):
            j = 4 * t + q
            p = q % 2
            np_ = 1 - p

            if q == 0:
                @pl.when(t > 0)
                def _():
                    wait_scatter((q - 1) % 4, np_)
            else:
                wait_scatter(q - 1, np_)

            @pl.when(j + 1 < CHUNKS)
            def _():
                wait_slab(j + 1, (q + 1) % 4)
                gather((q + 1) % 4, np_)

            wait_gather(q, p)
            scatter(q, p)

            @pl.when(j + 2 < CHUNKS)
            def _():
                load_slab(j + 2, (q + 2) % 4)
        return 0

    lax.fori_loop(0, CHUNKS // 4, quad_body, 0)

    wait_scatter((CHUNKS - 1) % 4, (CHUNKS - 1) % 2)

    plsc.subcore_barrier()
    pltpu.sync_copy(accum.at[pl.ds(s * ROWS_PT, ROWS_PT)],
                    out_hbm.at[c, pl.ds(s * ROWS_PT, ROWS_PT)])


def kernel(x, edge_index, W1, b1, W2, b2):
    eidx = jnp.stack([edge_index[1].reshape(NW, CHUNKS, K),
                      edge_index[0].reshape(NW, CHUNKS, K)], axis=2)
    zeros = jnp.zeros((N_PAD, D), jnp.float32)
    b1r = b1.reshape(1, D)
    b2r = b2.reshape(1, D)

    p = _sc_segment_sum(x, eidx, zeros)
    h = _tc_fused_mms(p, b1r, W1, W2)
    q = _sc_segment_sum(h, eidx, zeros)
    return _tc_combine(q, b2r)
